# baseline (speedup 1.0000x reference)
import math

import jax
import jax.numpy as jnp
from jax.experimental import pallas as pl

B, N, DIM, MAXB, HID = 8, 5000, 512, 3, 16
NBLK = 2560
NB = 2
NCAND = NB * MAXB
NEG = -1e30


def _score_block_kernel(text_ref, x_ref, cv_ref, ci_ref, cc_ref):
    k = pl.program_id(1)
    tf = text_ref[pl.ds(pl.program_id(0), 1), :]
    col = jax.lax.broadcasted_iota(jnp.int32, (1, NBLK), 1)
    n0 = k * NBLK
    valid = (n0 + col) < N
    feat = x_ref[0, 0]

    dot = jnp.dot(tf, feat, preferred_element_type=jnp.float32)
    ssq = jnp.sum(feat * feat, axis=0, keepdims=True)
    tnorm = jnp.sqrt(jnp.sum(tf * tf)) + 1e-8
    score = (100.0 * dot) / ((jnp.sqrt(ssq) + 1e-8) * tnorm)
    score = jnp.where(valid, score, NEG)

    vals, idxs = [], []
    cur = score
    for _ in range(MAXB):
        m = jnp.max(cur)
        i = jnp.min(jnp.where(cur == m, col, NBLK))
        vals.append(m)
        idxs.append(i)
        cur = jnp.where(col == i, NEG, cur)

    row3 = jax.lax.broadcasted_iota(jnp.int32, (MAXB, 1), 0)
    idx_mat = (idxs[0] * (row3 == 0) + idxs[1] * (row3 == 1)
               + idxs[2] * (row3 == 2))
    oh = (jax.lax.broadcasted_iota(jnp.int32, (MAXB, NBLK), 1)
          == idx_mat).astype(jnp.float32)

    @pl.when(k < NB - 1)
    def _():
        cc_ref[0, 0] = jax.lax.dot_general(
            oh, feat, (((1,), (1,)), ((), ())),
            preferred_element_type=jnp.float32)

    @pl.when(k == NB - 1)
    def _():
        featm = jnp.where(valid, feat, 0.0)
        cc_ref[0, 0] = jax.lax.dot_general(
            oh, featm, (((1,), (1,)), ((), ())),
            preferred_element_type=jnp.float32)

    lane = jax.lax.broadcasted_iota(jnp.int32, (1, 128), 1)
    vvec = jnp.full((1, 128), NEG, jnp.float32)
    ivec = jnp.zeros((1, 128), jnp.int32)
    for j in range(MAXB):
        vvec = jnp.where(lane == j, vals[j], vvec)
        ivec = jnp.where(lane == j, idxs[j] + n0, ivec)
    cv_ref[0, 0] = vvec
    ci_ref[0, 0] = ivec


def _epilogue_kernel(cv_ref, ci_ref, cc_ref, boxes_ref,
                     Wq_ref, bq_ref, Wk_ref, bk_ref, Wv_ref, bv_ref,
                     Wo_ref, bo_ref, W1_ref, b1_ref, g1_ref, be1_ref,
                     W2_ref, b2_ref, out_ref):
    R = B * MAXB
    rc_list, v_list, n_list = [], [], []
    for b in range(B):
        S = cv_ref[b].reshape(NB, 128)
        I = ci_ref[b].reshape(NB, 128)
        pos = (jax.lax.broadcasted_iota(jnp.int32, (NB, 128), 0) * 128
               + jax.lax.broadcasted_iota(jnp.int32, (NB, 128), 1))
        for _ in range(MAXB):
            m = jnp.max(S)
            p = jnp.min(jnp.where(S == m, pos, NB * 128))
            n_orig = jnp.sum(jnp.where(pos == p, I, 0))
            r = p // 128
            c = p - r * 128
            rc_list.append(b * NCAND + r * MAXB + c)
            v_list.append(m)
            n_list.append(n_orig)
            S = jnp.where(pos == p, NEG, S)

    C = cc_ref[...].reshape(B * NCAND, DIM)
    rowR = jax.lax.broadcasted_iota(jnp.int32, (R, 1), 0)
    rc_mat = jnp.zeros((R, 1), jnp.int32)
    v_mat = jnp.zeros((R, 1), jnp.float32)
    n_mat = jnp.zeros((R, 1), jnp.int32)
    for j in range(R):
        rc_mat = jnp.where(rowR == j, rc_list[j], rc_mat)
        v_mat = jnp.where(rowR == j, v_list[j], v_mat)
        n_mat = jnp.where(rowR == j, n_list[j], n_mat)
    OH = (jax.lax.broadcasted_iota(jnp.int32, (R, B * NCAND), 1)
          == rc_mat).astype(jnp.float32)
    H = jnp.dot(OH, C, preferred_element_type=jnp.float32)

    q = jnp.dot(H, Wq_ref[...], preferred_element_type=jnp.float32) + bq_ref[...]
    kk = jnp.dot(H, Wk_ref[...], preferred_element_type=jnp.float32) + bk_ref[...]
    v = jnp.dot(H, Wv_ref[...], preferred_element_type=jnp.float32) + bv_ref[...]
    logits = jax.lax.dot_general(
        q, kk, (((1,), (1,)), ((), ())),
        preferred_element_type=jnp.float32) / math.sqrt(float(DIM))
    ri = jax.lax.broadcasted_iota(jnp.int32, (R, R), 0)
    ci = jax.lax.broadcasted_iota(jnp.int32, (R, R), 1)
    same_b = (ri // MAXB) == (ci // MAXB)
    logits = jnp.where(same_b, logits, NEG)
    logits = logits - jnp.max(logits, axis=1, keepdims=True)
    e = jnp.exp(logits)
    attn = e / jnp.sum(e, axis=1, keepdims=True)
    gam = jnp.dot(jnp.dot(attn, v, preferred_element_type=jnp.float32),
                  Wo_ref[...], preferred_element_type=jnp.float32) + bo_ref[...]
    xs = (gam + v_mat).reshape(B, MAXB)

    col3 = jax.lax.broadcasted_iota(jnp.int32, (B, MAXB), 1)
    tvec = jnp.max(xs, axis=1, keepdims=True)
    jstar = jnp.min(jnp.where(xs == tvec, col3, MAXB), axis=1,
                    keepdims=True)
    NS = n_mat.reshape(B, MAXB)
    n_sel = jnp.sum(jnp.where(col3 == jstar, NS, 0), axis=1,
                    keepdims=True)

    r1 = tvec * W1_ref[...] + b1_ref[...]
    r1 = 0.5 * r1 * (1.0 + jax.lax.erf(r1 / math.sqrt(2.0)))
    mu = jnp.mean(r1, axis=1, keepdims=True)
    var = jnp.mean((r1 - mu) ** 2, axis=1, keepdims=True)
    r1 = (r1 - mu) / jnp.sqrt(var + 1e-5) * g1_ref[...] + be1_ref[...]
    r2 = jnp.dot(r1, W2_ref[...], preferred_element_type=jnp.float32)
    r2 = jnp.maximum(r2 + b2_ref[...], 0.0)

    boxes_sel = []
    for b in range(B):
        nb = jnp.sum(jnp.where(jax.lax.broadcasted_iota(jnp.int32, (B, 1), 0)
                               == b, n_sel, 0))
        boxes_sel.append(boxes_ref[b, pl.ds(nb, 1), :])
    out_ref[...] = r2 + jnp.concatenate(boxes_sel, axis=0)


@jax.jit
def kernel(text_feat, x, boxes, Wq, bq, Wk, bk, Wv, bv, Wo, bo,
           W1, b1, g1, be1, W2, b2):
    cv, ci, cc = pl.pallas_call(
        _score_block_kernel,
        grid=(B, NB),
        in_specs=[
            pl.BlockSpec((B, DIM), lambda b, k: (0, 0)),
            pl.BlockSpec((1, 1, DIM, NBLK), lambda b, k: (b, 0, 0, k)),
        ],
        out_specs=[
            pl.BlockSpec((1, 1, 1, 128), lambda b, k: (b, k, 0, 0)),
            pl.BlockSpec((1, 1, 1, 128), lambda b, k: (b, k, 0, 0)),
            pl.BlockSpec((1, 1, MAXB, DIM), lambda b, k: (b, k, 0, 0)),
        ],
        out_shape=[
            jax.ShapeDtypeStruct((B, NB, 1, 128), jnp.float32),
            jax.ShapeDtypeStruct((B, NB, 1, 128), jnp.int32),
            jax.ShapeDtypeStruct((B, NB, MAXB, DIM), jnp.float32),
        ],
    )(text_feat, x)

    out = pl.pallas_call(
        _epilogue_kernel,
        out_shape=jax.ShapeDtypeStruct((B, 4), jnp.float32),
    )(cv, ci, cc, boxes, Wq, bq.reshape(1, DIM), Wk, bk.reshape(1, DIM),
      Wv, bv.reshape(1, DIM), Wo, bo.reshape(1, 1), W1, b1.reshape(1, HID),
      g1.reshape(1, HID), be1.reshape(1, HID), W2, b2.reshape(1, 4))
    return out


# speedup vs baseline: 1.0456x; 1.0456x over previous
import math

import jax
import jax.numpy as jnp
from jax.experimental import pallas as pl
from jax.experimental.pallas import tpu as pltpu

B, N, DIM, MAXB, HID = 8, 5000, 512, 3, 16
NBLK = 2560
NB = 2
NCAND = NB * MAXB
NEG = -1e30


def _fused_kernel(text_ref, x_ref, boxes_ref,
                  Wq_ref, bq_ref, Wk_ref, bk_ref, Wv_ref, bv_ref,
                  Wo_ref, bo_ref, W1_ref, b1_ref, g1_ref, be1_ref,
                  W2_ref, b2_ref, out_ref, sv_ref, si_ref, sc_ref):
    b = pl.program_id(0)
    k = pl.program_id(1)
    tf = text_ref[pl.ds(b, 1), :]
    col = jax.lax.broadcasted_iota(jnp.int32, (1, NBLK), 1)
    n0 = k * NBLK
    valid = (n0 + col) < N
    feat = x_ref[0, 0]

    dot = jnp.dot(tf, feat, preferred_element_type=jnp.float32)
    ssq = jnp.sum(feat * feat, axis=0, keepdims=True)
    tnorm = jnp.sqrt(jnp.sum(tf * tf)) + 1e-8
    score = (100.0 * dot) / ((jnp.sqrt(ssq) + 1e-8) * tnorm)
    score = jnp.where(valid, score, NEG)

    vals, idxs = [], []
    cur = score
    for _ in range(MAXB):
        m = jnp.max(cur)
        i = jnp.min(jnp.where(cur == m, col, NBLK))
        vals.append(m)
        idxs.append(i)
        cur = jnp.where(col == i, NEG, cur)

    row8 = jax.lax.broadcasted_iota(jnp.int32, (8, 1), 0)
    idx_mat = (idxs[0] * (row8 == 0) + idxs[1] * (row8 == 1)
               + idxs[2] * (row8 == 2) - (row8 >= MAXB))
    oh = (jax.lax.broadcasted_iota(jnp.int32, (8, NBLK), 1)
          == idx_mat).astype(jnp.float32)
    cbase = (b * NB + k) * 8

    @pl.when(k < NB - 1)
    def _():
        sc_ref[pl.ds(cbase, 8), :] = jax.lax.dot_general(
            oh, feat, (((1,), (1,)), ((), ())),
            preferred_element_type=jnp.float32)

    @pl.when(k == NB - 1)
    def _():
        featm = jnp.where(valid, feat, 0.0)
        sc_ref[pl.ds(cbase, 8), :] = jax.lax.dot_general(
            oh, featm, (((1,), (1,)), ((), ())),
            preferred_element_type=jnp.float32)

    lane = jax.lax.broadcasted_iota(jnp.int32, (1, 128), 1)
    vvec = jnp.full((1, 128), NEG, jnp.float32)
    ivec = jnp.zeros((1, 128), jnp.int32)
    for j in range(MAXB):
        vvec = jnp.where(lane == j, vals[j], vvec)
        ivec = jnp.where(lane == j, idxs[j] + n0, ivec)
    sv_ref[pl.ds(b * 8, 1), pl.ds(k * 128, 128)] = vvec
    si_ref[pl.ds(b * 8, 1), pl.ds(k * 128, 128)] = ivec

    @pl.when((b == B - 1) & (k == NB - 1))
    def _epilogue():
        R = B * MAXB
        S = jnp.concatenate([sv_ref[bb * 8:bb * 8 + 1, :] for bb in range(B)],
                            axis=0)
        I = jnp.concatenate([si_ref[bb * 8:bb * 8 + 1, :] for bb in range(B)],
                            axis=0)
        pos = jax.lax.broadcasted_iota(jnp.int32, (B, NB * 128), 1)

        v_cols, p_cols, n_cols = [], [], []
        for _ in range(MAXB):
            m = jnp.max(S, axis=1, keepdims=True)
            p = jnp.min(jnp.where(S == m, pos, NB * 128),
                        axis=1, keepdims=True)
            n_orig = jnp.sum(jnp.where(pos == p, I, 0),
                             axis=1, keepdims=True)
            v_cols.append(m)
            p_cols.append(p)
            n_cols.append(n_orig)
            S = jnp.where(pos == p, NEG, S)
        v_b3 = jnp.concatenate(v_cols, axis=1)
        p_b3 = jnp.concatenate(p_cols, axis=1)
        n_b3 = jnp.concatenate(n_cols, axis=1)
        r_b3 = p_b3 // 128
        c_b3 = p_b3 - r_b3 * 128
        rowB = jax.lax.broadcasted_iota(jnp.int32, (B, MAXB), 0)
        rc_b3 = (rowB * NB + r_b3) * 8 + c_b3

        laneC = jax.lax.broadcasted_iota(jnp.int32, (B, B * NB * 8), 1)
        Hm = [jnp.dot((laneC == rc_b3[:, m:m + 1]).astype(jnp.float32),
                      sc_ref[...], preferred_element_type=jnp.float32)
              for m in range(MAXB)]
        Hcat = jnp.concatenate(Hm, axis=0)

        qc = (jnp.dot(Hcat, Wq_ref[...], preferred_element_type=jnp.float32)
              + bq_ref[...])
        kc = (jnp.dot(Hcat, Wk_ref[...], preferred_element_type=jnp.float32)
              + bk_ref[...])
        vc = (jnp.dot(Hcat, Wv_ref[...], preferred_element_type=jnp.float32)
              + bv_ref[...])
        wo = (jnp.dot(vc, Wo_ref[...], preferred_element_type=jnp.float32)
              + bo_ref[...])
        qm = [qc[8 * m:8 * (m + 1), :] for m in range(MAXB)]
        km = [kc[8 * m:8 * (m + 1), :] for m in range(MAXB)]
        wom = [wo[8 * m:8 * (m + 1), :] for m in range(MAXB)]
        inv_sqrt_d = 1.0 / math.sqrt(float(DIM))
        gam_cols = []
        for m in range(MAXB):
            L = jnp.concatenate(
                [jnp.sum(qm[m] * km[mp], axis=1, keepdims=True) * inv_sqrt_d
                 for mp in range(MAXB)], axis=1)
            L = L - jnp.max(L, axis=1, keepdims=True)
            e = jnp.exp(L)
            attn = e / jnp.sum(e, axis=1, keepdims=True)
            gam_cols.append(sum(attn[:, mp:mp + 1] * wom[mp]
                                for mp in range(MAXB)))
        xs = jnp.concatenate(gam_cols, axis=1) + v_b3

        col3 = jax.lax.broadcasted_iota(jnp.int32, (B, MAXB), 1)
        tvec = jnp.max(xs, axis=1, keepdims=True)
        jstar = jnp.min(jnp.where(xs == tvec, col3, MAXB),
                        axis=1, keepdims=True)
        n_sel = jnp.sum(jnp.where(col3 == jstar, n_b3, 0),
                        axis=1, keepdims=True)

        r1 = tvec * W1_ref[...] + b1_ref[...]
        r1 = 0.5 * r1 * (1.0 + jax.lax.erf(r1 / math.sqrt(2.0)))
        mu = jnp.mean(r1, axis=1, keepdims=True)
        var = jnp.mean((r1 - mu) ** 2, axis=1, keepdims=True)
        r1 = (r1 - mu) / jnp.sqrt(var + 1e-5) * g1_ref[...] + be1_ref[...]
        r2 = jnp.dot(r1, W2_ref[...], preferred_element_type=jnp.float32)
        r2 = jnp.maximum(r2 + b2_ref[...], 0.0)

        boxes_sel = []
        for bb in range(B):
            nb = jnp.sum(jnp.where(
                jax.lax.broadcasted_iota(jnp.int32, (B, 1), 0) == bb,
                n_sel, 0))
            boxes_sel.append(boxes_ref[bb, pl.ds(nb, 1), :])
        out_ref[...] = r2 + jnp.concatenate(boxes_sel, axis=0)


@jax.jit
def kernel(text_feat, x, boxes, Wq, bq, Wk, bk, Wv, bv, Wo, bo,
           W1, b1, g1, be1, W2, b2):
    rep = lambda b, k: (0, 0)
    out = pl.pallas_call(
        _fused_kernel,
        grid=(B, NB),
        in_specs=[
            pl.BlockSpec((B, DIM), rep),
            pl.BlockSpec((1, 1, DIM, NBLK), lambda b, k: (b, 0, 0, k)),
            pl.BlockSpec((B, N, 4), lambda b, k: (0, 0, 0)),
            pl.BlockSpec((DIM, DIM), rep),
            pl.BlockSpec((1, DIM), rep),
            pl.BlockSpec((DIM, DIM), rep),
            pl.BlockSpec((1, DIM), rep),
            pl.BlockSpec((DIM, DIM), rep),
            pl.BlockSpec((1, DIM), rep),
            pl.BlockSpec((DIM, 1), rep),
            pl.BlockSpec((1, 1), rep),
            pl.BlockSpec((1, HID), rep),
            pl.BlockSpec((1, HID), rep),
            pl.BlockSpec((1, HID), rep),
            pl.BlockSpec((1, HID), rep),
            pl.BlockSpec((HID, 4), rep),
            pl.BlockSpec((1, 4), rep),
        ],
        out_specs=pl.BlockSpec((B, 4), rep),
        out_shape=jax.ShapeDtypeStruct((B, 4), jnp.float32),
        scratch_shapes=[
            pltpu.VMEM((B * 8, NB * 128), jnp.float32),
            pltpu.VMEM((B * 8, NB * 128), jnp.int32),
            pltpu.VMEM((B * NB * 8, DIM), jnp.float32),
        ],
    )(text_feat, x, boxes, Wq, bq.reshape(1, DIM), Wk, bk.reshape(1, DIM),
      Wv, bv.reshape(1, DIM), Wo, bo.reshape(1, 1), W1, b1.reshape(1, HID),
      g1.reshape(1, HID), be1.reshape(1, HID), W2, b2.reshape(1, 4))
    return out
